# baseline (speedup 1.0000x reference)
import functools

import jax
import jax.numpy as jnp
from jax import lax
from jax.experimental import pallas as pl
from jax.experimental.pallas import tpu as pltpu
from jax.experimental.pallas import tpu_sc as plsc

_N = 50000
_S = 1024
_D = 8
_E = 1600000
_NC = 2
_NS = 16
_NW = _NC * _NS
_L = 16
_CH = 1280
_ROWS = _CH // 128
_NCHUNK = _E // _CH
_MAXG = (_NCHUNK + _NW - 1) // _NW
_MY = _S * _S // _NS
_ZCH = 2048

def _sc_body(si_hbm, sj_hbm, sidx_hbm, gam_hbm, del_hbm,
             out0_hbm, out1_hbm, gs_hbm, ds_hbm,
             pos_v, sia_v, sja_v, sib_v, sjb_v, idx_v, val_v, zer_v, m_sh,
             sem_ia, sem_ja, sem_ib, sem_jb, sem_z):
    cid = lax.axis_index("c")
    sid = lax.axis_index("s")
    wid = sid * _NC + cid

    pltpu.sync_copy(sidx_hbm, sia_v.at[pl.ds(0, _S)])

    @pl.when((sid == 0) & (cid == 0))
    def _():
        pltpu.async_copy(gam_hbm.at[sia_v.at[pl.ds(0, _S)]],
                         zer_v.at[pl.ds(0, _S)], sem_z).wait()
        pltpu.sync_copy(zer_v.at[pl.ds(0, _S)], gs_hbm)

    @pl.when((sid == 0) & (cid == 1))
    def _():
        pltpu.async_copy(del_hbm.at[sia_v.at[pl.ds(0, _S)]],
                         zer_v.at[pl.ds(0, _S)], sem_z).wait()
        pltpu.sync_copy(zer_v.at[pl.ds(0, _S)], ds_hbm)

    def _init(i, carry):
        pos_v[pl.ds(i * _L, _L)] = jnp.full((_L,), -1, jnp.int32)
        return carry
    lax.fori_loop(0, _N // _L, _init, 0, unroll=5)

    def _set(i, carry):
        nodes = sia_v[pl.ds(i * _L, _L)]
        slots = jnp.full((_L,), i * _L, jnp.int32) + lax.iota(jnp.int32, _L)
        plsc.store_scatter(pos_v, [nodes], slots)
        return carry
    lax.fori_loop(0, _S // _L, _set, 0)

    def _zb(i, carry):
        zer_v[pl.ds(i * _L, _L)] = jnp.zeros((_L,), jnp.float32)
        return carry
    lax.fori_loop(0, _ZCH // _L, _zb, 0)

    nz = _MY // _ZCH
    for r in range(nz):
        pltpu.async_copy(zer_v, m_sh.at[pl.ds(sid * _MY + r * _ZCH, _ZCH)],
                         sem_z)
    for r in range(nz):
        pltpu.make_async_copy(
            zer_v, m_sh.at[pl.ds(sid * _MY + r * _ZCH, _ZCH)], sem_z).wait()
    plsc.subcore_barrier()

    def _start(c, si_buf, sj_buf, sem_i, sem_j):
        base = c * _CH
        pltpu.async_copy(si_hbm.at[pl.ds(base, _CH)], si_buf, sem_i)
        pltpu.async_copy(sj_hbm.at[pl.ds(base, _CH)], sj_buf, sem_j)

    def _wait(c, si_buf, sj_buf, sem_i, sem_j):
        base = c * _CH
        pltpu.make_async_copy(si_hbm.at[pl.ds(base, _CH)], si_buf,
                              sem_i).wait()
        pltpu.make_async_copy(sj_hbm.at[pl.ds(base, _CH)], sj_buf,
                              sem_j).wait()

    def _process(si_buf, sj_buf):
        def _row(r, carry):
            acc = jnp.zeros((_L,), jnp.int32)
            for k in range(8):
                off = (r * 8 + k) * _L
                si = si_buf[pl.ds(off, _L)]
                sj = sj_buf[pl.ds(off, _L)]
                pi = plsc.load_gather(pos_v, [si])
                pj = plsc.load_gather(pos_v, [sj])
                valid = (pi >= 0) & (pj >= 0)
                vi = valid.astype(jnp.int32)
                flat = jnp.where(
                    valid,
                    (pj // 128) * (_S * 128) + pi * 128 + (pj % 128),
                    0)
                idx_v[r, pl.ds(k * _L, _L)] = flat
                val_v[r, pl.ds(k * _L, _L)] = vi.astype(jnp.float32)
                acc = acc | vi
            cnt = jnp.sum(acc, axis=0)

            @pl.when(cnt > 0)
            def _():
                pltpu.sync_copy(val_v.at[r], m_sh.at[idx_v.at[r]], add=True)
            return carry
        lax.fori_loop(0, _ROWS, _row, 0, unroll=2)

    _start(wid, sia_v, sja_v, sem_ia, sem_ja)
    _start(wid + _NW, sib_v, sjb_v, sem_ib, sem_jb)

    def _round(h, carry):
        cA = wid + _NW * (2 * h)
        cB = wid + _NW * (2 * h + 1)

        @pl.when(cA < _NCHUNK)
        def _():
            _wait(cA, sia_v, sja_v, sem_ia, sem_ja)
            _process(sia_v, sja_v)
            cA2 = cA + 2 * _NW

            @pl.when(cA2 < _NCHUNK)
            def _():
                _start(cA2, sia_v, sja_v, sem_ia, sem_ja)

        @pl.when(cB < _NCHUNK)
        def _():
            _wait(cB, sib_v, sjb_v, sem_ib, sem_jb)
            _process(sib_v, sjb_v)
            cB2 = cB + 2 * _NW

            @pl.when(cB2 < _NCHUNK)
            def _():
                _start(cB2, sib_v, sjb_v, sem_ib, sem_jb)
        return carry
    lax.fori_loop(0, _MAXG // 2, _round, 0)

    plsc.subcore_barrier()

    @pl.when(cid == 0)
    def _():
        pltpu.sync_copy(m_sh.at[pl.ds(sid * _MY, _MY)],
                        out0_hbm.at[pl.ds(sid * _MY, _MY)])

    @pl.when(cid == 1)
    def _():
        pltpu.sync_copy(m_sh.at[pl.ds(sid * _MY, _MY)],
                        out1_hbm.at[pl.ds(sid * _MY, _MY)])


@functools.cache
def _get_sc_build_m():
    mesh = plsc.VectorSubcoreMesh(core_axis_name="c", subcore_axis_name="s",
                                  num_cores=_NC, num_subcores=_NS)
    return pl.kernel(
        _sc_body,
        out_type=[jax.ShapeDtypeStruct((_S * _S,), jnp.float32),
                  jax.ShapeDtypeStruct((_S * _S,), jnp.float32),
                  jax.ShapeDtypeStruct((_S,), jnp.float32),
                  jax.ShapeDtypeStruct((_S,), jnp.float32)],
        mesh=mesh,
        scratch_types=[
            pltpu.VMEM((_N,), jnp.int32),
            pltpu.VMEM((_CH,), jnp.int32),
            pltpu.VMEM((_CH,), jnp.int32),
            pltpu.VMEM((_CH,), jnp.int32),
            pltpu.VMEM((_CH,), jnp.int32),
            pltpu.VMEM((_ROWS, 128), jnp.int32),
            pltpu.VMEM((_ROWS, 128), jnp.float32),
            pltpu.VMEM((_ZCH,), jnp.float32),
            pltpu.VMEM_SHARED((_S * _S,), jnp.float32),
            pltpu.SemaphoreType.DMA,
            pltpu.SemaphoreType.DMA,
            pltpu.SemaphoreType.DMA,
            pltpu.SemaphoreType.DMA,
            pltpu.SemaphoreType.DMA,
        ],
        compiler_params=pltpu.CompilerParams(needs_layout_passes=False),
    )


def _tc_body(zt_ref, gt_ref, r_ref, zsr_ref, wsr_ref, gcol_ref, drow_ref,
             m0_ref, m1_ref, out_ref):
    zt = zt_ref[...]
    ez = jnp.exp(zt)
    raa = ez / jnp.sum(ez, axis=0, keepdims=True)
    cg = raa * (1.0 / (1.0 + jnp.exp(-gt_ref[...])))
    colsum = jnp.sum(cg, axis=1, keepdims=True)
    bt = lax.dot_general(cg, raa, (((1,), (1,)), ((), ())),
                         preferred_element_type=jnp.float32,
                         precision=lax.Precision.HIGHEST)
    a = jnp.dot(bt / colsum, r_ref[...],
                preferred_element_type=jnp.float32,
                precision=lax.Precision.HIGHEST)

    def _rowsoftmax(x):
        mm = jnp.max(x, axis=1, keepdims=True)
        ee = jnp.exp(x - mm)
        return ee / jnp.sum(ee, axis=1, keepdims=True)

    zs = jnp.dot(_rowsoftmax(zsr_ref[...]), a,
                 preferred_element_type=jnp.float32,
                 precision=lax.Precision.HIGHEST)
    ws = jnp.dot(_rowsoftmax(wsr_ref[...]), a,
                 preferred_element_type=jnp.float32,
                 precision=lax.Precision.HIGHEST)
    zw = lax.dot_general(zs, ws, (((1,), (1,)), ((), ())),
                         preferred_element_type=jnp.float32, precision=lax.Precision.HIGHEST)
    sqz = jnp.sum(zs * zs, axis=1, keepdims=True)
    sqw = lax.dot_general(jnp.ones((1, _D), jnp.float32), ws * ws,
                          (((1,), (1,)), ((), ())),
                          preferred_element_type=jnp.float32, precision=lax.Precision.HIGHEST)
    dist = jnp.sqrt(jnp.maximum(sqz + sqw - 2.0 * zw, 0.0)) + 1e-6
    t = gcol_ref[...] + drow_ref[...] - dist
    et = jnp.exp(t)
    di = lax.broadcasted_iota(jnp.int32, (_S, _S), 0)
    dj = lax.broadcasted_iota(jnp.int32, (_S, _S), 1)
    offdiag = jnp.where(di == dj, 0.0, et)
    z2 = jnp.float32(0.0)
    for q in range(8):
        mq = m0_ref[q] + m1_ref[q]
        z2 = z2 + jnp.sum(mq * t[:, q * 128:(q + 1) * 128])
    out_ref[0, 0] = z2 - jnp.sum(offdiag)


def kernel(gamma_1, delta_1, latent_z1, G, R, weights_signed,
           sparse_i, sparse_j, sample_idx, up_i, up_j, epoch):
    sidx = sample_idx.astype(jnp.int32)
    si = sparse_i.astype(jnp.int32)
    sj = sparse_j.astype(jnp.int32)

    mp0, mp1, gs, ds = _get_sc_build_m()(si, sj, sidx, gamma_1, delta_1)
    m0 = mp0.reshape(8, _S, 128)
    m1 = mp1.reshape(8, _S, 128)

    zt = latent_z1.T
    gt = G.T
    zsr = latent_z1[sidx]
    wsr = latent_z1[_N + sidx]
    gcol = gs[:, None]
    drow = ds[None, :]

    out = pl.pallas_call(
        _tc_body,
        out_shape=jax.ShapeDtypeStruct((1, 1), jnp.float32),
        out_specs=pl.BlockSpec(memory_space=pltpu.SMEM),
    )(zt, gt, R.astype(jnp.float32), zsr, wsr, gcol, drow, m0, m1)
    return out[0, 0]


# speedup vs baseline: 1.0051x; 1.0051x over previous
import functools

import jax
import jax.numpy as jnp
from jax import lax
from jax.experimental import pallas as pl
from jax.experimental.pallas import tpu as pltpu
from jax.experimental.pallas import tpu_sc as plsc

_N = 50000
_S = 1024
_D = 8
_E = 1600000
_NC = 2
_NS = 16
_NW = _NC * _NS
_L = 16
_CH = 1280
_ROWS = _CH // 128
_NCHUNK = _E // _CH
_MAXG = (_NCHUNK + _NW - 1) // _NW
_MY = _S * _S // _NS
_ZCH = 2048

def _sc_body(si_hbm, sj_hbm, sidx_hbm, gam_hbm, del_hbm,
             out0_hbm, out1_hbm, gs_hbm, ds_hbm,
             pos_v, sia_v, sja_v, sib_v, sjb_v, idx_v, val_v, zer_v, m_sh,
             sem_ia, sem_ja, sem_ib, sem_jb, sem_z):
    cid = lax.axis_index("c")
    sid = lax.axis_index("s")
    wid = sid * _NC + cid

    pltpu.sync_copy(sidx_hbm, sia_v.at[pl.ds(0, _S)])

    @pl.when((sid == 0) & (cid == 0))
    def _():
        pltpu.async_copy(gam_hbm.at[sia_v.at[pl.ds(0, _S)]],
                         zer_v.at[pl.ds(0, _S)], sem_z).wait()
        pltpu.sync_copy(zer_v.at[pl.ds(0, _S)], gs_hbm)

    @pl.when((sid == 0) & (cid == 1))
    def _():
        pltpu.async_copy(del_hbm.at[sia_v.at[pl.ds(0, _S)]],
                         zer_v.at[pl.ds(0, _S)], sem_z).wait()
        pltpu.sync_copy(zer_v.at[pl.ds(0, _S)], ds_hbm)

    def _init(i, carry):
        pos_v[pl.ds(i * _L, _L)] = jnp.full((_L,), -1, jnp.int32)
        return carry
    lax.fori_loop(0, _N // _L, _init, 0, unroll=5)

    def _set(i, carry):
        nodes = sia_v[pl.ds(i * _L, _L)]
        slots = jnp.full((_L,), i * _L, jnp.int32) + lax.iota(jnp.int32, _L)
        plsc.store_scatter(pos_v, [nodes], slots)
        return carry
    lax.fori_loop(0, _S // _L, _set, 0)

    def _zb(i, carry):
        zer_v[pl.ds(i * _L, _L)] = jnp.zeros((_L,), jnp.float32)
        return carry
    lax.fori_loop(0, _ZCH // _L, _zb, 0)

    nz = _MY // _ZCH
    for r in range(nz):
        pltpu.async_copy(zer_v, m_sh.at[pl.ds(sid * _MY + r * _ZCH, _ZCH)],
                         sem_z)
    for r in range(nz):
        pltpu.make_async_copy(
            zer_v, m_sh.at[pl.ds(sid * _MY + r * _ZCH, _ZCH)], sem_z).wait()
    plsc.subcore_barrier()

    def _start(c, si_buf, sj_buf, sem_i, sem_j):
        base = c * _CH
        pltpu.async_copy(si_hbm.at[pl.ds(base, _CH)], si_buf, sem_i)
        pltpu.async_copy(sj_hbm.at[pl.ds(base, _CH)], sj_buf, sem_j)

    def _wait(c, si_buf, sj_buf, sem_i, sem_j):
        base = c * _CH
        pltpu.make_async_copy(si_hbm.at[pl.ds(base, _CH)], si_buf,
                              sem_i).wait()
        pltpu.make_async_copy(sj_hbm.at[pl.ds(base, _CH)], sj_buf,
                              sem_j).wait()

    def _process(si_buf, sj_buf):
        def _row(r, carry):
            acc = jnp.zeros((_L,), jnp.int32)
            for k in range(8):
                off = (r * 8 + k) * _L
                si = si_buf[pl.ds(off, _L)]
                sj = sj_buf[pl.ds(off, _L)]
                pi = plsc.load_gather(pos_v, [si])
                pj = plsc.load_gather(pos_v, [sj])
                valid = (pi >= 0) & (pj >= 0)
                vi = valid.astype(jnp.int32)
                flat = jnp.where(
                    valid,
                    (pj // 128) * (_S * 128) + pi * 128 + (pj % 128),
                    0)
                idx_v[r, pl.ds(k * _L, _L)] = flat
                val_v[r, pl.ds(k * _L, _L)] = vi.astype(jnp.float32)
                acc = acc | vi
            cnt = jnp.sum(acc, axis=0)

            @pl.when(cnt > 0)
            def _():
                pltpu.sync_copy(val_v.at[r], m_sh.at[idx_v.at[r]], add=True)
            return carry
        lax.fori_loop(0, _ROWS, _row, 0)

    _start(wid, sia_v, sja_v, sem_ia, sem_ja)
    _start(wid + _NW, sib_v, sjb_v, sem_ib, sem_jb)

    def _round(h, carry):
        cA = wid + _NW * (2 * h)
        cB = wid + _NW * (2 * h + 1)

        @pl.when(cA < _NCHUNK)
        def _():
            _wait(cA, sia_v, sja_v, sem_ia, sem_ja)
            _process(sia_v, sja_v)
            cA2 = cA + 2 * _NW

            @pl.when(cA2 < _NCHUNK)
            def _():
                _start(cA2, sia_v, sja_v, sem_ia, sem_ja)

        @pl.when(cB < _NCHUNK)
        def _():
            _wait(cB, sib_v, sjb_v, sem_ib, sem_jb)
            _process(sib_v, sjb_v)
            cB2 = cB + 2 * _NW

            @pl.when(cB2 < _NCHUNK)
            def _():
                _start(cB2, sib_v, sjb_v, sem_ib, sem_jb)
        return carry
    lax.fori_loop(0, _MAXG // 2, _round, 0)

    plsc.subcore_barrier()

    @pl.when(cid == 0)
    def _():
        pltpu.sync_copy(m_sh.at[pl.ds(sid * _MY, _MY)],
                        out0_hbm.at[pl.ds(sid * _MY, _MY)])

    @pl.when(cid == 1)
    def _():
        pltpu.sync_copy(m_sh.at[pl.ds(sid * _MY, _MY)],
                        out1_hbm.at[pl.ds(sid * _MY, _MY)])


@functools.cache
def _get_sc_build_m():
    mesh = plsc.VectorSubcoreMesh(core_axis_name="c", subcore_axis_name="s",
                                  num_cores=_NC, num_subcores=_NS)
    return pl.kernel(
        _sc_body,
        out_type=[jax.ShapeDtypeStruct((_S * _S,), jnp.float32),
                  jax.ShapeDtypeStruct((_S * _S,), jnp.float32),
                  jax.ShapeDtypeStruct((_S,), jnp.float32),
                  jax.ShapeDtypeStruct((_S,), jnp.float32)],
        mesh=mesh,
        scratch_types=[
            pltpu.VMEM((_N,), jnp.int32),
            pltpu.VMEM((_CH,), jnp.int32),
            pltpu.VMEM((_CH,), jnp.int32),
            pltpu.VMEM((_CH,), jnp.int32),
            pltpu.VMEM((_CH,), jnp.int32),
            pltpu.VMEM((_ROWS, 128), jnp.int32),
            pltpu.VMEM((_ROWS, 128), jnp.float32),
            pltpu.VMEM((_ZCH,), jnp.float32),
            pltpu.VMEM_SHARED((_S * _S,), jnp.float32),
            pltpu.SemaphoreType.DMA,
            pltpu.SemaphoreType.DMA,
            pltpu.SemaphoreType.DMA,
            pltpu.SemaphoreType.DMA,
            pltpu.SemaphoreType.DMA,
        ],
        compiler_params=pltpu.CompilerParams(needs_layout_passes=False),
    )


def _tc_body(zt_ref, gt_ref, r_ref, zsr_ref, wsr_ref, gcol_ref, drow_ref,
             m0_ref, m1_ref, out_ref):
    zt = zt_ref[...]
    ez = jnp.exp(zt)
    raa = ez / jnp.sum(ez, axis=0, keepdims=True)
    cg = raa * (1.0 / (1.0 + jnp.exp(-gt_ref[...])))
    colsum = jnp.sum(cg, axis=1, keepdims=True)
    bt = lax.dot_general(cg, raa, (((1,), (1,)), ((), ())),
                         preferred_element_type=jnp.float32,
                         precision=lax.Precision.HIGHEST)
    a = jnp.dot(bt / colsum, r_ref[...],
                preferred_element_type=jnp.float32,
                precision=lax.Precision.HIGHEST)

    def _rowsoftmax(x):
        mm = jnp.max(x, axis=1, keepdims=True)
        ee = jnp.exp(x - mm)
        return ee / jnp.sum(ee, axis=1, keepdims=True)

    zs = jnp.dot(_rowsoftmax(zsr_ref[...]), a,
                 preferred_element_type=jnp.float32,
                 precision=lax.Precision.HIGHEST)
    ws = jnp.dot(_rowsoftmax(wsr_ref[...]), a,
                 preferred_element_type=jnp.float32,
                 precision=lax.Precision.HIGHEST)
    zw = lax.dot_general(zs, ws, (((1,), (1,)), ((), ())),
                         preferred_element_type=jnp.float32, precision=lax.Precision.HIGHEST)
    sqz = jnp.sum(zs * zs, axis=1, keepdims=True)
    sqw = lax.dot_general(jnp.ones((1, _D), jnp.float32), ws * ws,
                          (((1,), (1,)), ((), ())),
                          preferred_element_type=jnp.float32, precision=lax.Precision.HIGHEST)
    dist = jnp.sqrt(jnp.maximum(sqz + sqw - 2.0 * zw, 0.0)) + 1e-6
    t = gcol_ref[...] + drow_ref[...] - dist
    et = jnp.exp(t)
    di = lax.broadcasted_iota(jnp.int32, (_S, _S), 0)
    dj = lax.broadcasted_iota(jnp.int32, (_S, _S), 1)
    offdiag = jnp.where(di == dj, 0.0, et)
    z2 = jnp.float32(0.0)
    for q in range(8):
        mq = m0_ref[q] + m1_ref[q]
        z2 = z2 + jnp.sum(mq * t[:, q * 128:(q + 1) * 128])
    out_ref[0, 0] = z2 - jnp.sum(offdiag)


def kernel(gamma_1, delta_1, latent_z1, G, R, weights_signed,
           sparse_i, sparse_j, sample_idx, up_i, up_j, epoch):
    sidx = sample_idx.astype(jnp.int32)
    si = sparse_i.astype(jnp.int32)
    sj = sparse_j.astype(jnp.int32)

    mp0, mp1, gs, ds = _get_sc_build_m()(si, sj, sidx, gamma_1, delta_1)
    m0 = mp0.reshape(8, _S, 128)
    m1 = mp1.reshape(8, _S, 128)

    zt = latent_z1.T
    gt = G.T
    zsr = latent_z1[sidx]
    wsr = latent_z1[_N + sidx]
    gcol = gs[:, None]
    drow = ds[None, :]

    out = pl.pallas_call(
        _tc_body,
        out_shape=jax.ShapeDtypeStruct((1, 1), jnp.float32),
        out_specs=pl.BlockSpec(memory_space=pltpu.SMEM),
    )(zt, gt, R.astype(jnp.float32), zsr, wsr, gcol, drow, m0, m1)
    return out[0, 0]
